# baseline (speedup 1.0000x reference)
import functools

import jax
import jax.numpy as jnp
from jax import lax
from jax.experimental import pallas as pl
from jax.experimental.pallas import tpu as pltpu
from jax.experimental.pallas import tpu_sc as plsc

B, S, D = 1, 2048, 768
H, E, K, P = 12, 8, 2, 64
HP = H * P
HEp = 128
EHP = E * H * P
SBLK = 256
NBLK = S // SBLK

_NEG = -1e30


def _rot_lanes(x, s):
    n = x.shape[-1]
    s = s % n
    if s == 0:
        return x
    return jnp.concatenate([x[:, s:], x[:, :s]], axis=1)


def _rot_group8(x, s, e_idx):
    a = _rot_lanes(x, s)
    b = _rot_lanes(x, s - 8)
    return jnp.where(e_idx < 8 - s, a, b)


def _group8_reduce(x, e_idx, op):
    for s in (4, 2, 1):
        x = op(x, _rot_group8(x, s, e_idx))
    return x


def _top2_gate(probs, e_idx):
    fmax = jnp.maximum
    imin = jnp.minimum
    m1 = _group8_reduce(probs, e_idx, fmax)
    cand1 = jnp.where(probs == m1, e_idx, 8)
    i1 = _group8_reduce(cand1, e_idx, imin)
    probs2 = jnp.where(e_idx == i1, jnp.full_like(probs, _NEG), probs)
    m2 = _group8_reduce(probs2, e_idx, fmax)
    cand2 = jnp.where(probs2 == m2, e_idx, 8)
    i2 = _group8_reduce(cand2, e_idx, imin)
    denom = fmax(m1 + m2, 1e-9)
    gate = jnp.where(e_idx == i1, m1, jnp.where(e_idx == i2, m2, 0.0))
    return gate / denom


def _proj_kernel(x_ref, wqk_ref, v2e_ref, selw_ref, rep_ref, q_ref, k_ref,
                 vmix_ref, go_ref):
    xb = x_ref[...]
    x16 = xb.astype(jnp.bfloat16)
    qk = jnp.dot(x16, wqk_ref[...], preferred_element_type=jnp.float32)
    q_ref[...] = qk[:, :HP].astype(jnp.bfloat16)
    k_ref[...] = qk[:, HP:].astype(jnp.bfloat16)
    logits = jnp.dot(xb, selw_ref[...], preferred_element_type=jnp.float32)
    e_idx = lax.broadcasted_iota(jnp.int32, (SBLK, HEp), 1) % 8
    probs_v = jax.nn.sigmoid(logits[:, :HEp])
    gate_v = _top2_gate(probs_v, e_idx)
    go_ref[...] = logits[:, HEp:]
    allv = jnp.dot(x16, v2e_ref[...], preferred_element_type=jnp.float32)
    gate_big = jnp.dot(gate_v.astype(jnp.bfloat16), rep_ref[...],
                       preferred_element_type=jnp.float32)
    prod = allv * gate_big
    acc = prod[:, :HP]
    for e in range(1, E):
        acc = acc + prod[:, e * HP:(e + 1) * HP]
    vmix_ref[...] = acc.astype(jnp.bfloat16)


_NW = 16
_TPW = S // _NW


def _sc_gate_kernel(lo_ref, out_ref, buf_in, buf_out):
    wid = lax.axis_index("s") * 2 + lax.axis_index("c")

    def vi(val):
        return jnp.full((16,), val, jnp.int32)

    def vf(val):
        return jnp.full((16,), val, jnp.float32)

    one = vf(1.0)
    zero = vf(0.0)
    neg = vf(_NEG)
    eps = vf(1e-9)
    e_vecs = [vi(e) for e in range(8)]

    @pl.when(wid < _NW)
    def _():
        base = wid * _TPW
        pltpu.sync_copy(lo_ref.at[:, pl.ds(base, _TPW)], buf_in)

        def chunk(ti, carry):
            t0 = ti * 16
            for g in range(HEp // 8):
                probs = []
                for e in range(8):
                    xv = buf_in[g * 8 + e, pl.ds(t0, 16)]
                    probs.append(one / (one + jnp.exp(-xv)))
                m1 = probs[0]
                for e in range(1, 8):
                    m1 = jnp.maximum(m1, probs[e])
                i1 = vi(8)
                for e in range(7, -1, -1):
                    i1 = jnp.where(probs[e] == m1, e_vecs[e], i1)
                m2 = neg
                for e in range(8):
                    m2 = jnp.maximum(m2,
                                     jnp.where(i1 == e_vecs[e], neg, probs[e]))
                i2 = vi(8)
                for e in range(7, -1, -1):
                    i2 = jnp.where((i1 != e_vecs[e]) & (probs[e] == m2),
                                   e_vecs[e], i2)
                denom = jnp.maximum(m1 + m2, eps)
                for e in range(8):
                    ge = jnp.where(i1 == e_vecs[e], m1,
                                   jnp.where(i2 == e_vecs[e], m2, zero)) / denom
                    buf_out[g * 8 + e, pl.ds(t0, 16)] = ge
            return carry

        lax.fori_loop(0, _TPW // 16, chunk, 0)
        pltpu.sync_copy(buf_out, out_ref.at[:, pl.ds(base, _TPW)])


def _attn_kernel(q_ref, k_ref, v_ref, o_ref, q_off=0):
    qi = pl.program_id(0) + q_off
    skv = k_ref.shape[0]
    row = qi * SBLK + lax.broadcasted_iota(jnp.int32, (SBLK, skv), 0)
    col = lax.broadcasted_iota(jnp.int32, (SBLK, skv), 1)
    causal = col <= row
    for h in range(H):
        sl = slice(h * P, (h + 1) * P)
        scores = lax.dot_general(q_ref[:, sl], k_ref[:, sl],
                                 (((1,), (1,)), ((), ())),
                                 preferred_element_type=jnp.float32)
        scores = jnp.where(causal, scores, _NEG)
        m = jnp.max(scores, axis=1, keepdims=True)
        p = jnp.exp(scores - m)
        denom = jnp.sum(p, axis=1, keepdims=True)
        acc = jnp.dot(p.astype(jnp.bfloat16), v_ref[:, sl],
                      preferred_element_type=jnp.float32)
        o_ref[:, sl] = (acc / denom).astype(jnp.bfloat16)


def _out_kernel(res_ref, go_ref, rep_ref, o2_ref, out_ref):
    res = res_ref[...]
    gate_big = jnp.dot(go_ref[...].astype(jnp.bfloat16), rep_ref[...],
                       preferred_element_type=jnp.float32)
    res8 = jnp.concatenate(
        [res[:, h * P:(h + 1) * P] for h in range(H) for _ in range(E)],
        axis=1)
    out_ref[...] = jnp.dot((res8 * gate_big).astype(jnp.bfloat16), o2_ref[...],
                           preferred_element_type=jnp.float32)


def kernel(x, Wq, Wk, v, o, sel_v, sel_o, route_scale):
    s = float(P) ** -0.25
    xf = x[0]
    pad = jnp.zeros((D, HEp - H * E), jnp.float32)
    wqk = jnp.concatenate([Wq.T * s, Wk.T * s], axis=1).astype(jnp.bfloat16)
    v2e = v.astype(jnp.bfloat16).reshape(H, E, D, P).transpose(2, 1, 0, 3)
    v2e = v2e.reshape(D, EHP)
    selw = jnp.concatenate([sel_v.T, pad, sel_o.T, pad], axis=1)

    r = jnp.arange(HEp)[:, None]
    c = jnp.arange(EHP)[None, :]
    real = r < H * E
    rep = (((r % 8) == (c // HP)) & ((r // 8) == ((c % HP) // P)) & real)
    rep = (rep.astype(jnp.float32) * route_scale[0]).astype(jnp.bfloat16)
    ep = E * P
    rep2 = (((r // 8) == (c // ep)) & ((r % 8) == ((c % ep) // P)) & real)
    rep2 = (rep2.astype(jnp.float32) * route_scale[0]).astype(jnp.bfloat16)

    q2, k2, vmix2, logits_o = pl.pallas_call(
        _proj_kernel,
        grid=(NBLK,),
        in_specs=[
            pl.BlockSpec((SBLK, D), lambda i: (i, 0)),
            pl.BlockSpec((D, 2 * HP), lambda i: (0, 0)),
            pl.BlockSpec((D, EHP), lambda i: (0, 0)),
            pl.BlockSpec((D, 2 * HEp), lambda i: (0, 0)),
            pl.BlockSpec((HEp, EHP), lambda i: (0, 0)),
        ],
        out_specs=[
            pl.BlockSpec((SBLK, HP), lambda i: (i, 0)),
            pl.BlockSpec((SBLK, HP), lambda i: (i, 0)),
            pl.BlockSpec((SBLK, HP), lambda i: (i, 0)),
            pl.BlockSpec((SBLK, HEp), lambda i: (i, 0)),
        ],
        out_shape=[
            jax.ShapeDtypeStruct((S, HP), jnp.bfloat16),
            jax.ShapeDtypeStruct((S, HP), jnp.bfloat16),
            jax.ShapeDtypeStruct((S, HP), jnp.bfloat16),
            jax.ShapeDtypeStruct((S, HEp), jnp.float32),
        ],
    )(xf, wqk, v2e, selw, rep)

    sc_gate = functools.partial(
        pl.kernel,
        mesh=plsc.VectorSubcoreMesh(core_axis_name="c", subcore_axis_name="s"),
        out_type=jax.ShapeDtypeStruct((HEp, S), jnp.float32),
        scratch_types=[
            pltpu.VMEM((HEp, _TPW), jnp.float32),
            pltpu.VMEM((HEp, _TPW), jnp.float32),
        ],
    )(_sc_gate_kernel)
    gate_o = sc_gate(logits_o.T).T

    nsplit = 4
    qspan = S // nsplit
    gsz = NBLK // nsplit
    rparts = []
    for si in range(nsplit):
        kv_len = (si + 1) * qspan
        rparts.append(pl.pallas_call(
            functools.partial(_attn_kernel, q_off=si * gsz),
            grid=(gsz,),
            in_specs=[
                pl.BlockSpec((SBLK, HP),
                             lambda i, si=si: (i + si * gsz, 0)),
                pl.BlockSpec((kv_len, HP), lambda i: (0, 0)),
                pl.BlockSpec((kv_len, HP), lambda i: (0, 0)),
            ],
            out_specs=pl.BlockSpec((SBLK, HP), lambda i: (i, 0)),
            out_shape=jax.ShapeDtypeStruct((qspan, HP), jnp.bfloat16),
        )(q2, k2, vmix2))

    res2 = jnp.concatenate(rparts, axis=0)

    o2e = o.astype(jnp.bfloat16).reshape(EHP, D)

    out = pl.pallas_call(
        _out_kernel,
        grid=(NBLK,),
        in_specs=[
            pl.BlockSpec((SBLK, HP), lambda i: (i, 0)),
            pl.BlockSpec((SBLK, HEp), lambda i: (i, 0)),
            pl.BlockSpec((HEp, EHP), lambda i: (0, 0)),
            pl.BlockSpec((EHP, D), lambda i: (0, 0)),
        ],
        out_specs=pl.BlockSpec((SBLK, D), lambda i: (i, 0)),
        out_shape=jax.ShapeDtypeStruct((S, D), jnp.float32),
    )(res2, gate_o, rep2, o2e)

    return out.reshape(B, S, D)


# speedup vs baseline: 1.0081x; 1.0081x over previous
import functools

import jax
import jax.numpy as jnp
from jax import lax
from jax.experimental import pallas as pl
from jax.experimental.pallas import tpu as pltpu
from jax.experimental.pallas import tpu_sc as plsc

B, S, D = 1, 2048, 768
H, E, K, P = 12, 8, 2, 64
HP = H * P
HEp = 128
EHP = E * H * P
SBLK = 256
NBLK = S // SBLK

_NEG = -1e30


def _rot_lanes(x, s):
    n = x.shape[-1]
    s = s % n
    if s == 0:
        return x
    return jnp.concatenate([x[:, s:], x[:, :s]], axis=1)


def _rot_group8(x, s, e_idx):
    a = _rot_lanes(x, s)
    b = _rot_lanes(x, s - 8)
    return jnp.where(e_idx < 8 - s, a, b)


def _group8_reduce(x, e_idx, op):
    for s in (4, 2, 1):
        x = op(x, _rot_group8(x, s, e_idx))
    return x


def _top2_gate(probs, e_idx):
    fmax = jnp.maximum
    imin = jnp.minimum
    m1 = _group8_reduce(probs, e_idx, fmax)
    cand1 = jnp.where(probs == m1, e_idx, 8)
    i1 = _group8_reduce(cand1, e_idx, imin)
    probs2 = jnp.where(e_idx == i1, jnp.full_like(probs, _NEG), probs)
    m2 = _group8_reduce(probs2, e_idx, fmax)
    cand2 = jnp.where(probs2 == m2, e_idx, 8)
    i2 = _group8_reduce(cand2, e_idx, imin)
    denom = fmax(m1 + m2, 1e-9)
    gate = jnp.where(e_idx == i1, m1, jnp.where(e_idx == i2, m2, 0.0))
    return gate / denom


def _proj_kernel(x_ref, wqk_ref, v2e_ref, selw_ref, rep_ref, q_ref, k_ref,
                 vmix_ref, go_ref):
    xb = x_ref[...]
    x16 = xb.astype(jnp.bfloat16)
    qk = jnp.dot(x16, wqk_ref[...], preferred_element_type=jnp.float32)
    q_ref[...] = qk[:, :HP].astype(jnp.bfloat16)
    k_ref[...] = qk[:, HP:].astype(jnp.bfloat16)
    logits = jnp.dot(xb, selw_ref[...], preferred_element_type=jnp.float32)
    e_idx = lax.broadcasted_iota(jnp.int32, (SBLK, HEp), 1) % 8
    probs_v = jax.nn.sigmoid(logits[:, :HEp])
    gate_v = _top2_gate(probs_v, e_idx)
    go_ref[...] = logits[:, HEp:]
    allv = jnp.dot(x16, v2e_ref[...], preferred_element_type=jnp.float32)
    gate_big = jnp.dot(gate_v.astype(jnp.bfloat16), rep_ref[...],
                       preferred_element_type=jnp.float32)
    prod = allv * gate_big
    acc = prod[:, :HP]
    for e in range(1, E):
        acc = acc + prod[:, e * HP:(e + 1) * HP]
    vmix_ref[...] = acc.astype(jnp.bfloat16)


_NW = 16
_TPW = S // _NW


def _sc_gate_kernel(lo_ref, out_ref, buf_in, buf_out):
    wid = lax.axis_index("s") * 2 + lax.axis_index("c")

    def vi(val):
        return jnp.full((16,), val, jnp.int32)

    def vf(val):
        return jnp.full((16,), val, jnp.float32)

    one = vf(1.0)
    zero = vf(0.0)
    neg = vf(_NEG)
    eps = vf(1e-9)
    e_vecs = [vi(e) for e in range(8)]

    @pl.when(wid < _NW)
    def _():
        base = wid * _TPW
        pltpu.sync_copy(lo_ref.at[:, pl.ds(base, _TPW)], buf_in)

        def chunk(ti, carry):
            t0 = ti * 16
            for g in range(HEp // 8):
                probs = []
                for e in range(8):
                    xv = buf_in[g * 8 + e, pl.ds(t0, 16)]
                    probs.append(one / (one + jnp.exp(-xv)))
                m1 = probs[0]
                for e in range(1, 8):
                    m1 = jnp.maximum(m1, probs[e])
                i1 = vi(8)
                for e in range(7, -1, -1):
                    i1 = jnp.where(probs[e] == m1, e_vecs[e], i1)
                m2 = neg
                for e in range(8):
                    m2 = jnp.maximum(m2,
                                     jnp.where(i1 == e_vecs[e], neg, probs[e]))
                i2 = vi(8)
                for e in range(7, -1, -1):
                    i2 = jnp.where((i1 != e_vecs[e]) & (probs[e] == m2),
                                   e_vecs[e], i2)
                denom = jnp.maximum(m1 + m2, eps)
                for e in range(8):
                    ge = jnp.where(i1 == e_vecs[e], m1,
                                   jnp.where(i2 == e_vecs[e], m2, zero)) / denom
                    buf_out[g * 8 + e, pl.ds(t0, 16)] = ge
            return carry

        lax.fori_loop(0, _TPW // 16, chunk, 0)
        pltpu.sync_copy(buf_out, out_ref.at[:, pl.ds(base, _TPW)])


def _attn_kernel(q_ref, k_ref, v_ref, o_ref, q_off=0):
    qi = pl.program_id(0) + q_off
    skv = k_ref.shape[0]
    row = qi * SBLK + lax.broadcasted_iota(jnp.int32, (SBLK, skv), 0)
    col = lax.broadcasted_iota(jnp.int32, (SBLK, skv), 1)
    causal = col <= row
    for h in range(H):
        sl = slice(h * P, (h + 1) * P)
        scores = lax.dot_general(q_ref[:, sl], k_ref[:, sl],
                                 (((1,), (1,)), ((), ())),
                                 preferred_element_type=jnp.float32)
        scores = jnp.where(causal, scores, _NEG)
        m = jnp.max(scores, axis=1, keepdims=True)
        p = jnp.exp(scores - m)
        denom = jnp.sum(p, axis=1, keepdims=True)
        acc = jnp.dot(p.astype(jnp.bfloat16), v_ref[:, sl],
                      preferred_element_type=jnp.float32)
        o_ref[:, sl] = (acc / denom).astype(jnp.bfloat16)


def _out_kernel(res_ref, go_ref, rep_ref, o2_ref, out_ref):
    res = res_ref[...]
    gate_big = jnp.dot(go_ref[...].astype(jnp.bfloat16), rep_ref[...],
                       preferred_element_type=jnp.float32)
    res8 = jnp.concatenate([res] * E, axis=1)
    out_ref[...] = jnp.dot((res8 * gate_big).astype(jnp.bfloat16), o2_ref[...],
                           preferred_element_type=jnp.float32)


def kernel(x, Wq, Wk, v, o, sel_v, sel_o, route_scale):
    s = float(P) ** -0.25
    xf = x[0]
    pad = jnp.zeros((D, HEp - H * E), jnp.float32)
    wqk = jnp.concatenate([Wq.T * s, Wk.T * s], axis=1).astype(jnp.bfloat16)
    v2e = v.astype(jnp.bfloat16).reshape(H, E, D, P).transpose(2, 1, 0, 3)
    v2e = v2e.reshape(D, EHP)
    selw = jnp.concatenate([sel_v.T, pad, sel_o.T, pad], axis=1)

    r = jnp.arange(HEp)[:, None]
    c = jnp.arange(EHP)[None, :]
    rep = (((r % 8) == (c // HP)) & ((r // 8) == ((c % HP) // P)) & (r < H * E))
    rep = rep.astype(jnp.float32) * route_scale[0]
    rep = rep.astype(jnp.bfloat16)

    q2, k2, vmix2, logits_o = pl.pallas_call(
        _proj_kernel,
        grid=(NBLK,),
        in_specs=[
            pl.BlockSpec((SBLK, D), lambda i: (i, 0)),
            pl.BlockSpec((D, 2 * HP), lambda i: (0, 0)),
            pl.BlockSpec((D, EHP), lambda i: (0, 0)),
            pl.BlockSpec((D, 2 * HEp), lambda i: (0, 0)),
            pl.BlockSpec((HEp, EHP), lambda i: (0, 0)),
        ],
        out_specs=[
            pl.BlockSpec((SBLK, HP), lambda i: (i, 0)),
            pl.BlockSpec((SBLK, HP), lambda i: (i, 0)),
            pl.BlockSpec((SBLK, HP), lambda i: (i, 0)),
            pl.BlockSpec((SBLK, HEp), lambda i: (i, 0)),
        ],
        out_shape=[
            jax.ShapeDtypeStruct((S, HP), jnp.bfloat16),
            jax.ShapeDtypeStruct((S, HP), jnp.bfloat16),
            jax.ShapeDtypeStruct((S, HP), jnp.bfloat16),
            jax.ShapeDtypeStruct((S, HEp), jnp.float32),
        ],
    )(xf, wqk, v2e, selw, rep)

    sc_gate = functools.partial(
        pl.kernel,
        mesh=plsc.VectorSubcoreMesh(core_axis_name="c", subcore_axis_name="s"),
        out_type=jax.ShapeDtypeStruct((HEp, S), jnp.float32),
        scratch_types=[
            pltpu.VMEM((HEp, _TPW), jnp.float32),
            pltpu.VMEM((HEp, _TPW), jnp.float32),
        ],
    )(_sc_gate_kernel)
    gate_o = sc_gate(logits_o.T).T

    nsplit = 4
    qspan = S // nsplit
    gsz = NBLK // nsplit
    rparts = []
    for si in range(nsplit):
        kv_len = (si + 1) * qspan
        rparts.append(pl.pallas_call(
            functools.partial(_attn_kernel, q_off=si * gsz),
            grid=(gsz,),
            in_specs=[
                pl.BlockSpec((SBLK, HP),
                             lambda i, si=si: (i + si * gsz, 0)),
                pl.BlockSpec((kv_len, HP), lambda i: (0, 0)),
                pl.BlockSpec((kv_len, HP), lambda i: (0, 0)),
            ],
            out_specs=pl.BlockSpec((SBLK, HP), lambda i: (i, 0)),
            out_shape=jax.ShapeDtypeStruct((qspan, HP), jnp.bfloat16),
        )(q2, k2, vmix2))

    res2 = jnp.concatenate(rparts, axis=0)

    o2e = o.astype(jnp.bfloat16).reshape(H, E, P, D).transpose(1, 0, 2, 3)
    o2e = o2e.reshape(EHP, D)

    out = pl.pallas_call(
        _out_kernel,
        grid=(NBLK,),
        in_specs=[
            pl.BlockSpec((SBLK, HP), lambda i: (i, 0)),
            pl.BlockSpec((SBLK, HEp), lambda i: (i, 0)),
            pl.BlockSpec((HEp, EHP), lambda i: (0, 0)),
            pl.BlockSpec((EHP, D), lambda i: (0, 0)),
        ],
        out_specs=pl.BlockSpec((SBLK, D), lambda i: (i, 0)),
        out_shape=jax.ShapeDtypeStruct((S, D), jnp.float32),
    )(res2, gate_o, rep, o2e)

    return out.reshape(B, S, D)
